# Initial kernel scaffold; baseline (speedup 1.0000x reference)
import jax
import jax.numpy as jnp
from jax.experimental import pallas as pl


def kernel(x, grid):
    raise NotImplementedError("write your pallas kernel here")


# speedup vs baseline: 3.0008x; 3.0008x over previous
import functools

import jax
import jax.numpy as jnp
from jax import lax
from jax.experimental import pallas as pl
from jax.experimental.pallas import tpu as pltpu
from jax.experimental.pallas import tpu_sc as plsc

L = 16
NC = 2
NS = 16
NW = NC * NS
CHUNK = 128

_SPLAT_DNUMS = lax.GatherDimensionNumbers(
    offset_dims=(), collapsed_slice_dims=(0,), start_index_map=(0,))


def _splat(vec, lane):
    idx = jnp.full((L, 1), lane, jnp.int32)
    return lax.gather(vec, idx, _SPLAT_DNUMS, slice_sizes=(1,),
                      mode=lax.GatherScatterMode.PROMISE_IN_BOUNDS)


@functools.lru_cache(maxsize=None)
def _build(n_pts: int, C: int, R: int):
    assert n_pts % (NW * CHUNK) == 0
    assert C == 2 * L
    npw = n_pts // NW
    nchunk = npw // CHUNK
    groups = CHUNK // L
    scale = 0.5 * (R - 1)

    mesh = plsc.VectorSubcoreMesh(core_axis_name="c", subcore_axis_name="s")

    scratch = (
        [pltpu.VMEM((CHUNK,), jnp.float32) for _ in range(3)]
        + [pltpu.VMEM((CHUNK,), jnp.int32) for _ in range(8)]
        + [pltpu.VMEM((CHUNK,), jnp.float32) for _ in range(8)]
        + [pltpu.VMEM((CHUNK, C), jnp.float32) for _ in range(8)]
        + [pltpu.VMEM((CHUNK, C), jnp.float32)]
        + [pltpu.SemaphoreType.DMA]
    )

    @functools.partial(
        pl.kernel,
        mesh=mesh,
        out_type=jax.ShapeDtypeStruct((n_pts, C), jnp.float32),
        scratch_types=scratch,
        compiler_params=pltpu.CompilerParams(use_tc_tiling_on_sc=False),
    )
    def grid_sample_kernel(xs, ys, zs, table, out, *sc):
        cx, cy, cz = sc[0:3]
        idx = sc[3:11]
        wts = sc[11:19]
        rows = sc[19:27]
        oblk = sc[27]
        sem = sc[28]

        wid = lax.axis_index("s") * NC + lax.axis_index("c")
        base0 = wid * npw

        def chunk_body(j, carry):
            base = base0 + j * CHUNK
            pltpu.sync_copy(xs.at[pl.ds(base, CHUNK)], cx)
            pltpu.sync_copy(ys.at[pl.ds(base, CHUNK)], cy)
            pltpu.sync_copy(zs.at[pl.ds(base, CHUNK)], cz)

            def idx_body(g, carry2):
                s = g * L
                ix = (cx[pl.ds(s, L)] + 1.0) * scale
                iy = (cy[pl.ds(s, L)] + 1.0) * scale
                iz = (cz[pl.ds(s, L)] + 1.0) * scale
                x0 = jnp.clip(ix.astype(jnp.int32), 0, R - 2)
                y0 = jnp.clip(iy.astype(jnp.int32), 0, R - 2)
                z0 = jnp.clip(iz.astype(jnp.int32), 0, R - 2)
                fx = ix - x0.astype(jnp.float32)
                fy = iy - y0.astype(jnp.float32)
                fz = iz - z0.astype(jnp.float32)
                bx = 1.0 - fx
                by = 1.0 - fy
                bz = 1.0 - fz
                b000 = (z0 * R + y0) * R + x0
                idx[0][pl.ds(s, L)] = b000
                idx[1][pl.ds(s, L)] = b000 + 1
                idx[2][pl.ds(s, L)] = b000 + R
                idx[3][pl.ds(s, L)] = b000 + (R + 1)
                idx[4][pl.ds(s, L)] = b000 + R * R
                idx[5][pl.ds(s, L)] = b000 + (R * R + 1)
                idx[6][pl.ds(s, L)] = b000 + (R * R + R)
                idx[7][pl.ds(s, L)] = b000 + (R * R + R + 1)
                wts[0][pl.ds(s, L)] = bx * by * bz
                wts[1][pl.ds(s, L)] = fx * by * bz
                wts[2][pl.ds(s, L)] = bx * fy * bz
                wts[3][pl.ds(s, L)] = fx * fy * bz
                wts[4][pl.ds(s, L)] = bx * by * fz
                wts[5][pl.ds(s, L)] = fx * by * fz
                wts[6][pl.ds(s, L)] = bx * fy * fz
                wts[7][pl.ds(s, L)] = fx * fy * fz
                return carry2

            lax.fori_loop(0, groups, idx_body, 0)

            copies = [
                pltpu.async_copy(table.at[idx[k]], rows[k], sem)
                for k in range(8)
            ]
            for cp in copies:
                cp.wait()

            def blend_body(g, carry2):
                s = g * L
                wv = [wts[k][pl.ds(s, L)] for k in range(8)]
                for p_in in range(L):
                    p = s + p_in
                    w0 = _splat(wv[0], p_in)
                    acc_lo = w0 * rows[0][p, pl.ds(0, L)]
                    acc_hi = w0 * rows[0][p, pl.ds(L, L)]
                    for k in range(1, 8):
                        wk = _splat(wv[k], p_in)
                        acc_lo = acc_lo + wk * rows[k][p, pl.ds(0, L)]
                        acc_hi = acc_hi + wk * rows[k][p, pl.ds(L, L)]
                    oblk[p, pl.ds(0, L)] = acc_lo
                    oblk[p, pl.ds(L, L)] = acc_hi
                return carry2

            lax.fori_loop(0, groups, blend_body, 0)
            pltpu.sync_copy(oblk, out.at[pl.ds(base, CHUNK)])
            return carry

        lax.fori_loop(0, nchunk, chunk_body, 0)

    return grid_sample_kernel


def kernel(x, grid):
    C, D, H, W = grid.shape
    assert D == H == W
    pts = x.reshape(-1, 3)
    n = pts.shape[0]
    table = grid.reshape(C, D * H * W).T
    out = _build(n, C, D)(pts[:, 0], pts[:, 1], pts[:, 2], table)
    return out.reshape(tuple(x.shape[:-1]) + (C,))


# speedup vs baseline: 3.7581x; 1.2524x over previous
import functools

import jax
import jax.numpy as jnp
from jax import lax
from jax.experimental import pallas as pl
from jax.experimental.pallas import tpu as pltpu
from jax.experimental.pallas import tpu_sc as plsc

L = 16
NC = 2
NS = 16
NW = NC * NS
CHUNK = 128

_SPLAT_DNUMS = lax.GatherDimensionNumbers(
    offset_dims=(), collapsed_slice_dims=(0,), start_index_map=(0,))


def _splat(vec, lane):
    idx = jnp.full((L, 1), lane, jnp.int32)
    return lax.gather(vec, idx, _SPLAT_DNUMS, slice_sizes=(1,),
                      mode=lax.GatherScatterMode.PROMISE_IN_BOUNDS)


@functools.lru_cache(maxsize=None)
def _build(n_pts: int, C: int, R: int):
    assert n_pts % (NW * CHUNK) == 0
    assert C == 2 * L
    npw = n_pts // NW
    nchunk = npw // CHUNK
    assert nchunk % 2 == 0
    groups = CHUNK // L
    scale = 0.5 * (R - 1)

    mesh = plsc.VectorSubcoreMesh(core_axis_name="c", subcore_axis_name="s")

    per_parity = (
        [pltpu.VMEM((CHUNK,), jnp.float32) for _ in range(3)]
        + [pltpu.VMEM((CHUNK,), jnp.int32) for _ in range(8)]
        + [pltpu.VMEM((CHUNK,), jnp.float32) for _ in range(8)]
        + [pltpu.VMEM((CHUNK, C), jnp.float32) for _ in range(8)]
        + [pltpu.VMEM((CHUNK, C), jnp.float32)]
    )
    scratch = per_parity * 2 + [pltpu.SemaphoreType.DMA] * 6

    @functools.partial(
        pl.kernel,
        mesh=mesh,
        out_type=jax.ShapeDtypeStruct((n_pts, C), jnp.float32),
        scratch_types=scratch,
        compiler_params=pltpu.CompilerParams(use_tc_tiling_on_sc=False),
    )
    def grid_sample_kernel(xs, ys, zs, table, out, *sc):
        P = 28
        bufs = []
        for p in (0, 1):
            o = p * P
            bufs.append(dict(
                c=sc[o:o + 3], idx=sc[o + 3:o + 11], wts=sc[o + 11:o + 19],
                rows=sc[o + 19:o + 27], oblk=sc[o + 27]))
        semc = sc[2 * P:2 * P + 2]
        semg = sc[2 * P + 2:2 * P + 4]
        semo = sc[2 * P + 4:2 * P + 6]

        wid = lax.axis_index("s") * NC + lax.axis_index("c")
        base0 = wid * npw

        def coord_fire(j, p):
            base = base0 + j * CHUNK
            for src, dst in zip((xs, ys, zs), bufs[p]["c"]):
                pltpu.async_copy(src.at[pl.ds(base, CHUNK)], dst, semc[p])

        def coord_wait(j, p):
            base = base0 + j * CHUNK
            for src, dst in zip((xs, ys, zs), bufs[p]["c"]):
                pltpu.make_async_copy(src.at[pl.ds(base, CHUNK)], dst,
                                      semc[p]).wait()

        def comp_idx(p):
            cx, cy, cz = bufs[p]["c"]
            idx = bufs[p]["idx"]
            wts = bufs[p]["wts"]

            def idx_body(g, carry):
                s = g * L
                ix = (cx[pl.ds(s, L)] + 1.0) * scale
                iy = (cy[pl.ds(s, L)] + 1.0) * scale
                iz = (cz[pl.ds(s, L)] + 1.0) * scale
                x0 = jnp.clip(ix.astype(jnp.int32), 0, R - 2)
                y0 = jnp.clip(iy.astype(jnp.int32), 0, R - 2)
                z0 = jnp.clip(iz.astype(jnp.int32), 0, R - 2)
                fx = ix - x0.astype(jnp.float32)
                fy = iy - y0.astype(jnp.float32)
                fz = iz - z0.astype(jnp.float32)
                bx = 1.0 - fx
                by = 1.0 - fy
                bz = 1.0 - fz
                b000 = (z0 * R + y0) * R + x0
                idx[0][pl.ds(s, L)] = b000
                idx[1][pl.ds(s, L)] = b000 + 1
                idx[2][pl.ds(s, L)] = b000 + R
                idx[3][pl.ds(s, L)] = b000 + (R + 1)
                idx[4][pl.ds(s, L)] = b000 + R * R
                idx[5][pl.ds(s, L)] = b000 + (R * R + 1)
                idx[6][pl.ds(s, L)] = b000 + (R * R + R)
                idx[7][pl.ds(s, L)] = b000 + (R * R + R + 1)
                wts[0][pl.ds(s, L)] = bx * by * bz
                wts[1][pl.ds(s, L)] = fx * by * bz
                wts[2][pl.ds(s, L)] = bx * fy * bz
                wts[3][pl.ds(s, L)] = fx * fy * bz
                wts[4][pl.ds(s, L)] = bx * by * fz
                wts[5][pl.ds(s, L)] = fx * by * fz
                wts[6][pl.ds(s, L)] = bx * fy * fz
                wts[7][pl.ds(s, L)] = fx * fy * fz
                return carry

            lax.fori_loop(0, groups, idx_body, 0)

        def gather_fire(p):
            for k in range(8):
                pltpu.async_copy(table.at[bufs[p]["idx"][k]],
                                 bufs[p]["rows"][k], semg[p])

        def gather_wait(p):
            for k in range(8):
                pltpu.make_async_copy(table.at[bufs[p]["idx"][k]],
                                      bufs[p]["rows"][k], semg[p]).wait()

        def blend(p):
            rows = bufs[p]["rows"]
            wts = bufs[p]["wts"]
            oblk = bufs[p]["oblk"]

            def blend_body(g, carry):
                s = g * L
                wv = [wts[k][pl.ds(s, L)] for k in range(8)]
                for p_in in range(L):
                    q = s + p_in
                    w0 = _splat(wv[0], p_in)
                    acc_lo = w0 * rows[0][q, pl.ds(0, L)]
                    acc_hi = w0 * rows[0][q, pl.ds(L, L)]
                    for k in range(1, 8):
                        wk = _splat(wv[k], p_in)
                        acc_lo = acc_lo + wk * rows[k][q, pl.ds(0, L)]
                        acc_hi = acc_hi + wk * rows[k][q, pl.ds(L, L)]
                    oblk[q, pl.ds(0, L)] = acc_lo
                    oblk[q, pl.ds(L, L)] = acc_hi
                return carry

            lax.fori_loop(0, groups, blend_body, 0)

        def out_fire(j, p):
            base = base0 + j * CHUNK
            pltpu.async_copy(bufs[p]["oblk"], out.at[pl.ds(base, CHUNK)],
                             semo[p])

        def out_wait(j, p):
            base = base0 + j * CHUNK
            pltpu.make_async_copy(bufs[p]["oblk"], out.at[pl.ds(base, CHUNK)],
                                  semo[p]).wait()

        def out_prime(p):
            pltpu.async_copy(out.at[pl.ds(base0, CHUNK)], bufs[p]["oblk"],
                             semo[p])

        out_prime(0)
        out_prime(1)
        coord_fire(0, 0)
        coord_fire(1, 1)
        coord_wait(0, 0)
        comp_idx(0)
        gather_fire(0)
        coord_wait(1, 1)
        comp_idx(1)
        gather_fire(1)

        def chunk_pair(t, carry):
            for p in (0, 1):
                j = 2 * t + p
                gather_wait(p)
                out_wait(j, p)
                blend(p)
                out_fire(j, p)
                jn = jnp.minimum(j + 2, nchunk - 1)
                coord_fire(jn, p)
                coord_wait(jn, p)
                comp_idx(p)
                gather_fire(p)
            return carry

        lax.fori_loop(0, nchunk // 2, chunk_pair, 0)

        gather_wait(0)
        gather_wait(1)
        out_wait(nchunk - 2, 0)
        out_wait(nchunk - 1, 1)

    return grid_sample_kernel


def kernel(x, grid):
    C, D, H, W = grid.shape
    assert D == H == W
    pts = x.reshape(-1, 3)
    n = pts.shape[0]
    table = grid.reshape(C, D * H * W).T
    out = _build(n, C, D)(pts[:, 0], pts[:, 1], pts[:, 2], table)
    return out.reshape(tuple(x.shape[:-1]) + (C,))


# speedup vs baseline: 3.8661x; 1.0287x over previous
import functools

import jax
import jax.numpy as jnp
from jax import lax
from jax.experimental import pallas as pl
from jax.experimental.pallas import tpu as pltpu
from jax.experimental.pallas import tpu_sc as plsc

L = 16
NC = 2
NS = 16
NW = NC * NS
CHUNK = 128

_SPLAT_DNUMS = lax.GatherDimensionNumbers(
    offset_dims=(), collapsed_slice_dims=(0,), start_index_map=(0,))


def _splat(vec, lane):
    idx = jnp.full((L, 1), lane, jnp.int32)
    return lax.gather(vec, idx, _SPLAT_DNUMS, slice_sizes=(1,),
                      mode=lax.GatherScatterMode.PROMISE_IN_BOUNDS)


@functools.lru_cache(maxsize=None)
def _build_repack(C: int, R: int):
    V = R * R * R
    B = R * R

    def body(x_ref, o_ref):
        x = x_ref[...].reshape(C, B)
        y = x.T
        z = y.reshape(B // 4, 4, C)
        o_ref[...] = jnp.concatenate(
            [z[:, 0], z[:, 1], z[:, 2], z[:, 3]], axis=1)

    return pl.pallas_call(
        body,
        grid=(R,),
        in_specs=[pl.BlockSpec((C, 1, R, R), lambda i: (0, i, 0, 0))],
        out_specs=pl.BlockSpec((B * C // 128, 128), lambda i: (i, 0)),
        out_shape=jax.ShapeDtypeStruct((V * C // 128, 128), jnp.float32),
    )


@functools.lru_cache(maxsize=None)
def _build(n_pts: int, C: int, R: int):
    assert n_pts % (NW * CHUNK) == 0
    assert C == 2 * L
    npw = n_pts // NW
    nchunk = npw // CHUNK
    assert nchunk % 2 == 0
    groups = CHUNK // L
    scale = 0.5 * (R - 1)

    mesh = plsc.VectorSubcoreMesh(core_axis_name="c", subcore_axis_name="s")

    per_parity = (
        [pltpu.VMEM((CHUNK,), jnp.float32) for _ in range(3)]
        + [pltpu.VMEM((CHUNK,), jnp.int32) for _ in range(8)]
        + [pltpu.VMEM((CHUNK,), jnp.float32) for _ in range(8)]
        + [pltpu.VMEM((CHUNK, C), jnp.float32) for _ in range(8)]
        + [pltpu.VMEM((CHUNK, C), jnp.float32)]
    )
    scratch = per_parity * 2 + [pltpu.SemaphoreType.DMA] * 6

    @functools.partial(
        pl.kernel,
        mesh=mesh,
        out_type=jax.ShapeDtypeStruct((n_pts, C), jnp.float32),
        scratch_types=scratch,
        compiler_params=pltpu.CompilerParams(use_tc_tiling_on_sc=False),
    )
    def grid_sample_kernel(xs, ys, zs, table, out, *sc):
        P = 28
        bufs = []
        for p in (0, 1):
            o = p * P
            bufs.append(dict(
                c=sc[o:o + 3], idx=sc[o + 3:o + 11], wts=sc[o + 11:o + 19],
                rows=sc[o + 19:o + 27], oblk=sc[o + 27]))
        semc = sc[2 * P:2 * P + 2]
        semg = sc[2 * P + 2:2 * P + 4]
        semo = sc[2 * P + 4:2 * P + 6]

        wid = lax.axis_index("s") * NC + lax.axis_index("c")
        base0 = wid * npw

        def coord_fire(j, p):
            base = base0 + j * CHUNK
            for src, dst in zip((xs, ys, zs), bufs[p]["c"]):
                pltpu.async_copy(src.at[pl.ds(base, CHUNK)], dst, semc[p])

        def coord_wait(j, p):
            base = base0 + j * CHUNK
            for src, dst in zip((xs, ys, zs), bufs[p]["c"]):
                pltpu.make_async_copy(src.at[pl.ds(base, CHUNK)], dst,
                                      semc[p]).wait()

        def comp_idx(p):
            cx, cy, cz = bufs[p]["c"]
            idx = bufs[p]["idx"]
            wts = bufs[p]["wts"]

            def idx_body(g, carry):
                s = g * L
                ix = (cx[pl.ds(s, L)] + 1.0) * scale
                iy = (cy[pl.ds(s, L)] + 1.0) * scale
                iz = (cz[pl.ds(s, L)] + 1.0) * scale
                x0 = jnp.clip(ix.astype(jnp.int32), 0, R - 2)
                y0 = jnp.clip(iy.astype(jnp.int32), 0, R - 2)
                z0 = jnp.clip(iz.astype(jnp.int32), 0, R - 2)
                fx = ix - x0.astype(jnp.float32)
                fy = iy - y0.astype(jnp.float32)
                fz = iz - z0.astype(jnp.float32)
                bx = 1.0 - fx
                by = 1.0 - fy
                bz = 1.0 - fz
                b000 = (z0 * R + y0) * R + x0
                idx[0][pl.ds(s, L)] = b000
                idx[1][pl.ds(s, L)] = b000 + 1
                idx[2][pl.ds(s, L)] = b000 + R
                idx[3][pl.ds(s, L)] = b000 + (R + 1)
                idx[4][pl.ds(s, L)] = b000 + R * R
                idx[5][pl.ds(s, L)] = b000 + (R * R + 1)
                idx[6][pl.ds(s, L)] = b000 + (R * R + R)
                idx[7][pl.ds(s, L)] = b000 + (R * R + R + 1)
                wts[0][pl.ds(s, L)] = bx * by * bz
                wts[1][pl.ds(s, L)] = fx * by * bz
                wts[2][pl.ds(s, L)] = bx * fy * bz
                wts[3][pl.ds(s, L)] = fx * fy * bz
                wts[4][pl.ds(s, L)] = bx * by * fz
                wts[5][pl.ds(s, L)] = fx * by * fz
                wts[6][pl.ds(s, L)] = bx * fy * fz
                wts[7][pl.ds(s, L)] = fx * fy * fz
                return carry

            lax.fori_loop(0, groups, idx_body, 0)

        def gather_fire(p):
            for k in range(8):
                pltpu.async_copy(table.at[bufs[p]["idx"][k]],
                                 bufs[p]["rows"][k], semg[p])

        def gather_wait(p):
            for k in range(8):
                pltpu.make_async_copy(table.at[bufs[p]["idx"][k]],
                                      bufs[p]["rows"][k], semg[p]).wait()

        def blend(p):
            rows = bufs[p]["rows"]
            wts = bufs[p]["wts"]
            oblk = bufs[p]["oblk"]

            def blend_body(g, carry):
                s = g * L
                wv = [wts[k][pl.ds(s, L)] for k in range(8)]
                for p_in in range(L):
                    q = s + p_in
                    w0 = _splat(wv[0], p_in)
                    acc_lo = w0 * rows[0][q, pl.ds(0, L)]
                    acc_hi = w0 * rows[0][q, pl.ds(L, L)]
                    for k in range(1, 8):
                        wk = _splat(wv[k], p_in)
                        acc_lo = acc_lo + wk * rows[k][q, pl.ds(0, L)]
                        acc_hi = acc_hi + wk * rows[k][q, pl.ds(L, L)]
                    oblk[q, pl.ds(0, L)] = acc_lo
                    oblk[q, pl.ds(L, L)] = acc_hi
                return carry

            lax.fori_loop(0, groups, blend_body, 0)

        def out_fire(j, p):
            base = base0 + j * CHUNK
            pltpu.async_copy(bufs[p]["oblk"], out.at[pl.ds(base, CHUNK)],
                             semo[p])

        def out_wait(j, p):
            base = base0 + j * CHUNK
            pltpu.make_async_copy(bufs[p]["oblk"], out.at[pl.ds(base, CHUNK)],
                                  semo[p]).wait()

        def out_prime(p):
            pltpu.async_copy(out.at[pl.ds(base0, CHUNK)], bufs[p]["oblk"],
                             semo[p])

        out_prime(0)
        out_prime(1)
        coord_fire(0, 0)
        coord_fire(1, 1)
        coord_wait(0, 0)
        comp_idx(0)
        gather_fire(0)
        coord_wait(1, 1)
        comp_idx(1)
        gather_fire(1)

        def chunk_pair(t, carry):
            for p in (0, 1):
                j = 2 * t + p
                gather_wait(p)
                out_wait(j, p)
                blend(p)
                out_fire(j, p)
                jn = jnp.minimum(j + 2, nchunk - 1)
                coord_fire(jn, p)
                coord_wait(jn, p)
                comp_idx(p)
                gather_fire(p)
            return carry

        lax.fori_loop(0, nchunk // 2, chunk_pair, 0)

        gather_wait(0)
        gather_wait(1)
        out_wait(nchunk - 2, 0)
        out_wait(nchunk - 1, 1)

    return grid_sample_kernel


def kernel(x, grid):
    C, D, H, W = grid.shape
    assert D == H == W
    pts = x.reshape(-1, 3)
    n = pts.shape[0]
    V = D * H * W
    table = _build_repack(C, D)(grid).reshape(V, C)
    out = _build(n, C, D)(pts[:, 0], pts[:, 1], pts[:, 2], table)
    return out.reshape(tuple(x.shape[:-1]) + (C,))
